# Initial kernel scaffold; baseline (speedup 1.0000x reference)
import jax
import jax.numpy as jnp
from jax.experimental import pallas as pl


def kernel(x, neighbours, w, b):
    raise NotImplementedError("write your pallas kernel here")


# speedup vs baseline: 30.7264x; 30.7264x over previous
import functools

import jax
import jax.numpy as jnp
from jax import lax
from jax.experimental import pallas as pl
from jax.experimental.pallas import tpu as pltpu
from jax.experimental.pallas import tpu_sc as plsc

BATCH, NPIX, CIN, COUT, KS = 2, 196608, 16, 16, 9
KC = KS * COUT

PT = 2048


def _tc_body(x_ref, w2_ref, z_ref):
    w2 = w2_ref[...]
    z_ref[0] = jnp.dot(x_ref[0], w2, preferred_element_type=jnp.float32)
    z_ref[1] = jnp.dot(x_ref[1], w2, preferred_element_type=jnp.float32)


def _tc_stage(x, w2):
    return pl.pallas_call(
        _tc_body,
        grid=(NPIX // PT,),
        in_specs=[
            pl.BlockSpec((BATCH, PT, CIN), lambda i: (0, i, 0)),
            pl.BlockSpec((CIN, KC), lambda i: (0, 0)),
        ],
        out_specs=pl.BlockSpec((BATCH, PT, KC), lambda i: (0, i, 0)),
        out_shape=jax.ShapeDtypeStruct((BATCH, NPIX, KC), jnp.float32),
    )(x, w2)


NC, NS, L = 2, 16, 16
NW = NC * NS
PPW = NPIX // NW
CH = 128
NCHUNK = PPW // CH
NJ = BATCH * KS


@functools.cache
def _get_sc_stage():
    mesh = plsc.VectorSubcoreMesh(core_axis_name="c", subcore_axis_name="s")

    @functools.partial(
        pl.kernel,
        mesh=mesh,
        out_type=jax.ShapeDtypeStruct((BATCH, NPIX, COUT), jnp.float32),
        scratch_types=[
            pltpu.VMEM((NJ, CH), jnp.int32),
            pltpu.VMEM((NJ, CH, COUT), jnp.float32),
            pltpu.VMEM((CH, COUT), jnp.float32),
            pltpu.VMEM((CH, COUT), jnp.float32),
            pltpu.VMEM((L,), jnp.float32),
            pltpu.SemaphoreType.DMA,
        ],
        compiler_params=pltpu.CompilerParams(use_tc_tiling_on_sc=False),
    )
    def _sc_stage(z_hbm, gidx_hbm, bias_hbm, out_hbm, idx_v, gbuf, ob0, ob1,
                  bias_v, sem):
        wid = lax.axis_index("s") * NC + lax.axis_index("c")
        pltpu.sync_copy(bias_hbm, bias_v)
        bvec = bias_v[...]
        base0 = wid * PPW

        def chunk_body(ci, carry):
            base = base0 + ci * CH
            for j in range(NJ):
                pltpu.sync_copy(gidx_hbm.at[pl.ds(j * NPIX + base, CH)],
                                idx_v.at[j])
            copies = [
                pltpu.async_copy(z_hbm.at[idx_v.at[j]], gbuf.at[j], sem)
                for j in range(NJ)
            ]
            for c in copies:
                c.wait()

            def px_body(i, c2):
                a0 = bvec
                a1 = bvec
                for k in range(KS):
                    a0 = a0 + gbuf[k, i, :]
                    a1 = a1 + gbuf[KS + k, i, :]
                ob0[i, :] = a0
                ob1[i, :] = a1
                return c2

            lax.fori_loop(0, CH, px_body, 0)
            pltpu.sync_copy(ob0, out_hbm.at[0, pl.ds(base, CH)])
            pltpu.sync_copy(ob1, out_hbm.at[1, pl.ds(base, CH)])
            return carry

        lax.fori_loop(0, NCHUNK, chunk_body, 0)

    return _sc_stage


def kernel(x, neighbours, w, b):
    w2 = jnp.transpose(w, (2, 1, 0)).reshape(CIN, KC)
    z = _tc_stage(x, w2)
    zflat = z.reshape(BATCH * NPIX * KS, COUT)
    boffs = (jnp.arange(BATCH, dtype=jnp.int32) * NPIX)[:, None, None]
    koffs = jnp.arange(KS, dtype=jnp.int32)[None, :, None]
    gidx = ((neighbours.T[None] + boffs) * KS + koffs).reshape(NJ * NPIX)
    return _get_sc_stage()(zflat, gidx, b)
